# baseline (speedup 1.0000x reference)
import functools
import jax
import jax.numpy as jnp
from jax.experimental import pallas as pl
from jax.experimental.pallas import tpu as pltpu

N = 10000
E = 320000
H = 256
G = 128
BN_N = 1000
BE = 2560
NEG_INF = -jnp.inf


def _bdot(a, b):
    return jnp.dot(a.astype(jnp.bfloat16), b.astype(jnp.bfloat16),
                   preferred_element_type=jnp.float32)


def _embed_body(x_ref, wemb_ref, bemb_ref, w1a0_ref, h_ref, hw_ref):
    h = _bdot(x_ref[...], wemb_ref[...])
    h = h + bemb_ref[...]
    h_ref[...] = h
    hw = _bdot(h, w1a0_ref[...])
    hw_ref[0] = hw[:, :128]
    hw_ref[1] = hw[:, 128:]


def _embed(x, W_emb, b_emb, W1a0):
    nb = N // BN_N
    return pl.pallas_call(
        _embed_body,
        grid=(nb,),
        in_specs=[
            pl.BlockSpec((BN_N, 9), lambda i: (i, 0)),
            pl.BlockSpec((9, H), lambda i: (0, 0)),
            pl.BlockSpec((1, H), lambda i: (0, 0)),
            pl.BlockSpec((H, H), lambda i: (0, 0)),
        ],
        out_specs=[
            pl.BlockSpec((BN_N, H), lambda i: (i, 0)),
            pl.BlockSpec((2, BN_N, 128), lambda i: (0, i, 0)),
        ],
        out_shape=[
            jax.ShapeDtypeStruct((N, H), jnp.float32),
            jax.ShapeDtypeStruct((2, N, 128), jnp.float32),
        ],
    )(x, W_emb, b_emb.reshape(1, H), W1a0)


def _e1_body(ea_ref, w1b_ref, b1_ref, e1_ref):
    ea = ea_ref[...].astype(jnp.bfloat16).astype(jnp.float32)
    w = w1b_ref[...].astype(jnp.bfloat16).astype(jnp.float32)
    acc = b1_ref[...]
    e1 = (ea[:, 0:1] * w[0:1, :] + ea[:, 1:2] * w[1:2, :]
          + ea[:, 2:3] * w[2:3, :] + ea[:, 3:4] * w[3:4, :] + acc)
    e1_ref[0] = e1[:, :128]
    e1_ref[1] = e1[:, 128:]


def _e1(edge_attr, W1b_l, b1_l):
    nb = E // BE
    return pl.pallas_call(
        _e1_body,
        grid=(nb,),
        in_specs=[
            pl.BlockSpec((BE, 4), lambda i: (i, 0)),
            pl.BlockSpec((4, H), lambda i: (0, 0)),
            pl.BlockSpec((1, H), lambda i: (0, 0)),
        ],
        out_specs=pl.BlockSpec((2, BE, 128), lambda i: (0, i, 0)),
        out_shape=jax.ShapeDtypeStruct((2, E, 128), jnp.float32),
    )(edge_attr, W1b_l, b1_l.reshape(1, H))


def _gru_body(s_ref, deg_ref, h_ref, w2_ref, b2_ref, wih_ref, bih_ref,
              whh_ref, bhh_ref, hn_ref, stats_ref):
    i = pl.program_id(0)
    s0 = s_ref[0]
    s1 = s_ref[1]
    w2 = w2_ref[...].astype(jnp.bfloat16).astype(jnp.float32)
    aggr = jnp.dot(s0, w2[:128, :], preferred_element_type=jnp.float32,
                   precision=jax.lax.Precision.HIGHEST)
    aggr = aggr + jnp.dot(s1, w2[128:, :], preferred_element_type=jnp.float32,
                          precision=jax.lax.Precision.HIGHEST)
    aggr = aggr + deg_ref[...] * b2_ref[...]
    h = h_ref[...]
    gi = _bdot(aggr, wih_ref[...]) + bih_ref[...]
    gh = _bdot(h, whh_ref[...]) + bhh_ref[...]
    r = jax.nn.sigmoid(gi[:, :H] + gh[:, :H])
    z = jax.nn.sigmoid(gi[:, H:2 * H] + gh[:, H:2 * H])
    n = jnp.tanh(gi[:, 2 * H:] + r * gh[:, 2 * H:])
    hn = (1.0 - z) * n + z * h
    hn_ref[...] = hn

    @pl.when(i == 0)
    def _():
        stats_ref[...] = jnp.zeros_like(stats_ref)

    stats_ref[0:1, :] += jnp.sum(hn, axis=0, keepdims=True)
    stats_ref[1:2, :] += jnp.sum(hn * hn, axis=0, keepdims=True)


def _gru(s, deg, h, W2_l, b2_l, W_ihT, b_ih_l, W_hhT, b_hh_l):
    nb = N // BN_N
    return pl.pallas_call(
        _gru_body,
        grid=(nb,),
        in_specs=[
            pl.BlockSpec((2, BN_N, 128), lambda i: (0, i, 0)),
            pl.BlockSpec((BN_N, 1), lambda i: (i, 0)),
            pl.BlockSpec((BN_N, H), lambda i: (i, 0)),
            pl.BlockSpec((H, H), lambda i: (0, 0)),
            pl.BlockSpec((1, H), lambda i: (0, 0)),
            pl.BlockSpec((H, 3 * H), lambda i: (0, 0)),
            pl.BlockSpec((1, 3 * H), lambda i: (0, 0)),
            pl.BlockSpec((H, 3 * H), lambda i: (0, 0)),
            pl.BlockSpec((1, 3 * H), lambda i: (0, 0)),
        ],
        out_specs=[
            pl.BlockSpec((BN_N, H), lambda i: (i, 0)),
            pl.BlockSpec((2, H), lambda i: (0, 0)),
        ],
        out_shape=[
            jax.ShapeDtypeStruct((N, H), jnp.float32),
            jax.ShapeDtypeStruct((2, H), jnp.float32),
        ],
    )(s, deg, h, W2_l, b2_l.reshape(1, H), W_ihT, b_ih_l.reshape(1, 3 * H),
      W_hhT, b_hh_l.reshape(1, 3 * H))


def _bn_body(hn_ref, stats_ref, gam_ref, bet_ref, w1a_ref, h_ref, hw_ref, *,
             with_hw):
    hn = hn_ref[...]
    mu = stats_ref[0:1, :] * (1.0 / N)
    var = stats_ref[1:2, :] * (1.0 / N) - mu * mu
    inv = jax.lax.rsqrt(var + 1e-5)
    hnew = jnp.maximum((hn - mu) * inv * gam_ref[...] + bet_ref[...], 0.0)
    h_ref[...] = hnew
    if with_hw:
        hw = _bdot(hnew, w1a_ref[...])
        hw_ref[0] = hw[:, :128]
        hw_ref[1] = hw[:, 128:]


def _bn_relu(hn, stats, gamma_l, beta_l, W1a_next):
    nb = N // BN_N
    with_hw = W1a_next is not None
    w1a = W1a_next if with_hw else jnp.zeros((8, H), jnp.float32)
    out_specs = [pl.BlockSpec((BN_N, H), lambda i: (i, 0))]
    out_shape = [jax.ShapeDtypeStruct((N, H), jnp.float32)]
    if with_hw:
        out_specs.append(pl.BlockSpec((2, BN_N, 128), lambda i: (0, i, 0)))
        out_shape.append(jax.ShapeDtypeStruct((2, N, 128), jnp.float32))
    else:
        out_specs.append(pl.BlockSpec((1, 8), lambda i: (0, 0)))
        out_shape.append(jax.ShapeDtypeStruct((1, 8), jnp.float32))
    res = pl.pallas_call(
        functools.partial(_bn_body, with_hw=with_hw),
        grid=(nb,),
        in_specs=[
            pl.BlockSpec((BN_N, H), lambda i: (i, 0)),
            pl.BlockSpec((2, H), lambda i: (0, 0)),
            pl.BlockSpec((1, H), lambda i: (0, 0)),
            pl.BlockSpec((1, H), lambda i: (0, 0)),
            pl.BlockSpec(w1a.shape, lambda i: (0, 0)),
        ],
        out_specs=out_specs,
        out_shape=out_shape,
    )(hn, stats, gamma_l.reshape(1, H), beta_l.reshape(1, H), w1a)
    return res


def _pool_body(h_ref, ids_ref, mean_ref, max_ref, sum_sc, cnt_sc, max_sc, oh_sc):
    i = pl.program_id(0)
    nb = pl.num_programs(0)

    @pl.when(i == 0)
    def _():
        sum_sc[...] = jnp.zeros_like(sum_sc)
        cnt_sc[...] = jnp.zeros_like(cnt_sc)
        max_sc[...] = jnp.full_like(max_sc, NEG_INF)

    h = h_ref[...]
    ids = ids_ref[...]
    giota = jax.lax.broadcasted_iota(jnp.int32, (1, G), 1)
    onehot = (ids == giota).astype(jnp.float32)
    oh_sc[...] = onehot
    sum_sc[...] += jax.lax.dot_general(
        onehot, h, (((0,), (0,)), ((), ())), preferred_element_type=jnp.float32, precision=jax.lax.Precision.HIGHEST)
    cnt_sc[...] += jnp.sum(onehot, axis=0, keepdims=True)

    RB = 8
    def body(r, acc):
        oh = oh_sc[pl.ds(r * RB, RB), :]
        hc = h_ref[pl.ds(r * RB, RB), :]
        m3 = jnp.where(oh[:, :, None] > 0.0, hc[:, None, :], NEG_INF)
        return jnp.maximum(acc, jnp.max(m3, axis=0))
    mx = jax.lax.fori_loop(0, BN_N // RB, body, max_sc[...])
    max_sc[...] = mx

    @pl.when(i == nb - 1)
    def _():
        cnt = jnp.maximum(cnt_sc[...], 1.0)
        mean_ref[...] = sum_sc[...] / cnt.reshape(G, 1)
        max_ref[...] = max_sc[...]


def _pool(h, batch):
    nb = N // BN_N
    return pl.pallas_call(
        _pool_body,
        grid=(nb,),
        in_specs=[
            pl.BlockSpec((BN_N, H), lambda i: (i, 0)),
            pl.BlockSpec((BN_N, 1), lambda i: (i, 0)),
        ],
        out_specs=[
            pl.BlockSpec((G, H), lambda i: (0, 0)),
            pl.BlockSpec((G, H), lambda i: (0, 0)),
        ],
        out_shape=[
            jax.ShapeDtypeStruct((G, H), jnp.float32),
            jax.ShapeDtypeStruct((G, H), jnp.float32),
        ],
        scratch_shapes=[
            pltpu.VMEM((G, H), jnp.float32),
            pltpu.VMEM((1, G), jnp.float32),
            pltpu.VMEM((G, H), jnp.float32),
            pltpu.VMEM((BN_N, G), jnp.float32),
        ],
    )(h, batch.reshape(N, 1))


def _edge_phase(hw, e1, src, dst):
    p0 = jnp.maximum(hw[0][src] + e1[0], 0.0).astype(jnp.bfloat16).astype(jnp.float32)
    p1 = jnp.maximum(hw[1][src] + e1[1], 0.0).astype(jnp.bfloat16).astype(jnp.float32)
    s0 = jax.ops.segment_sum(p0, dst, num_segments=N)
    s1 = jax.ops.segment_sum(p1, dst, num_segments=N)
    return jnp.stack([s0, s1])


def kernel(x, edge_index, edge_attr, batch, W_emb, b_emb, W1, b1, W2, b2,
           W_ih, W_hh, b_ih, b_hh, bn_gamma, bn_beta):
    L = W1.shape[0]
    src = edge_index[0]
    dst = edge_index[1]
    W1a = W1[:, :H, :]
    W1b = W1[:, H:, :]
    deg = jax.ops.segment_sum(jnp.ones((E,), jnp.float32), dst,
                              num_segments=N).reshape(N, 1)

    h, hw = _embed(x, W_emb, b_emb, W1a[0])
    for l in range(L):
        e1 = _e1(edge_attr, W1b[l], b1[l])
        s = _edge_phase(hw, e1, src, dst)
        hn, stats = _gru(s, deg, h, W2[l], b2[l], W_ih[l].T, b_ih[l],
                         W_hh[l].T, b_hh[l])
        nxt = W1a[l + 1] if l + 1 < L else None
        res = _bn_relu(hn, stats, bn_gamma[l], bn_beta[l], nxt)
        h = res[0]
        if l + 1 < L:
            hw = res[1]
    x_mean, x_max = _pool(h, batch)
    return jnp.concatenate([x_mean, x_max], axis=-1)
